# Initial kernel scaffold; baseline (speedup 1.0000x reference)
import jax
import jax.numpy as jnp
from jax.experimental import pallas as pl


def kernel(embs, sample, w_relation):
    raise NotImplementedError("write your pallas kernel here")


# speedup vs baseline: 1.1695x; 1.1695x over previous
import functools

import jax
import jax.numpy as jnp
from jax import lax
from jax.experimental import pallas as pl
from jax.experimental.pallas import tpu as pltpu
from jax.experimental.pallas import tpu_sc as plsc

H_DIM = 128
L = 16
NC, NS = 2, 16
NW = NC * NS
B = 16384
B_PER_W = B // NW
CH = 128
NCH = B_PER_W // CH
NJ = H_DIM // L

_mesh = plsc.VectorSubcoreMesh(core_axis_name="c", subcore_axis_name="s",
                               num_cores=NC, num_subcores=NS)


@functools.partial(
    pl.kernel,
    out_type=jax.ShapeDtypeStruct((NW, B_PER_W), jnp.float32),
    mesh=_mesh,
    compiler_params=pltpu.CompilerParams(needs_layout_passes=False),
    scratch_types=[
        pltpu.VMEM((NCH, CH), jnp.int32),
        pltpu.VMEM((NCH, CH), jnp.int32),
        pltpu.VMEM((NCH, CH), jnp.int32),
        pltpu.VMEM((CH, H_DIM), jnp.float32),
        pltpu.VMEM((CH, H_DIM), jnp.float32),
        pltpu.VMEM((CH, H_DIM), jnp.float32),
        pltpu.VMEM((B_PER_W,), jnp.float32),
        pltpu.SemaphoreType.DMA,
    ],
)
def _score_kernel(embs_hbm, wrel_hbm, hidx_hbm, ridx_hbm, tidx_hbm, out_hbm,
                  hidx_v, ridx_v, tidx_v, h_v, r_v, t_v, out_v, sem):
    wid = lax.axis_index("s") * NC + lax.axis_index("c")
    pltpu.sync_copy(hidx_hbm.at[wid], hidx_v)
    pltpu.sync_copy(ridx_hbm.at[wid], ridx_v)
    pltpu.sync_copy(tidx_hbm.at[wid], tidx_v)

    lane_iota = lax.iota(jnp.int32, L)

    for c in range(NCH):
        cp_h = pltpu.async_copy(embs_hbm.at[hidx_v.at[c]], h_v, sem)
        cp_r = pltpu.async_copy(wrel_hbm.at[ridx_v.at[c]], r_v, sem)
        cp_t = pltpu.async_copy(embs_hbm.at[tidx_v.at[c]], t_v, sem)
        cp_h.wait()
        cp_r.wait()
        cp_t.wait()

        def group(g, _, c=c):
            res = jnp.zeros((L,), jnp.float32)
            for rr in range(L):
                b = g * L + rr
                acc = (h_v[b, pl.ds(0, L)] * r_v[b, pl.ds(0, L)]
                       * t_v[b, pl.ds(0, L)])
                for j in range(1, NJ):
                    acc = acc + (h_v[b, pl.ds(j * L, L)]
                                 * r_v[b, pl.ds(j * L, L)]
                                 * t_v[b, pl.ds(j * L, L)])
                res = jnp.where(lane_iota == rr, jnp.sum(acc), res)
            out_v[pl.ds(c * CH + g * L, L)] = res
            return 0

        lax.fori_loop(0, CH // L, group, 0, unroll=False)

    pltpu.sync_copy(out_v, out_hbm.at[wid])


def kernel(embs, sample, w_relation):
    sample = sample.astype(jnp.int32)
    hidx = sample[0].reshape(NW, NCH, CH)
    ridx = sample[1].reshape(NW, NCH, CH)
    tidx = sample[2].reshape(NW, NCH, CH)
    out = _score_kernel(embs, w_relation, hidx, ridx, tidx)
    return out.reshape(B, 1)


# speedup vs baseline: 2.2197x; 1.8980x over previous
import functools

import jax
import jax.numpy as jnp
from jax import lax
from jax.experimental import pallas as pl
from jax.experimental.pallas import tpu as pltpu
from jax.experimental.pallas import tpu_sc as plsc

H_DIM = 128
L = 16
NC, NS = 2, 16
NW = NC * NS
B = 16384
B_PER_W = B // NW
CH = 64
NCH = B_PER_W // CH
NJ = H_DIM // L

_mesh = plsc.VectorSubcoreMesh(core_axis_name="c", subcore_axis_name="s",
                               num_cores=NC, num_subcores=NS)


@functools.partial(
    pl.kernel,
    out_type=jax.ShapeDtypeStruct((NW, B_PER_W), jnp.float32),
    mesh=_mesh,
    compiler_params=pltpu.CompilerParams(needs_layout_passes=False),
    scratch_types=[
        pltpu.VMEM((NCH, CH), jnp.int32),
        pltpu.VMEM((NCH, CH), jnp.int32),
        pltpu.VMEM((NCH, CH), jnp.int32),
        pltpu.VMEM((2, CH, H_DIM), jnp.float32),
        pltpu.VMEM((2, CH, H_DIM), jnp.float32),
        pltpu.VMEM((2, CH, H_DIM), jnp.float32),
        pltpu.VMEM((B_PER_W + L,), jnp.float32),
        pltpu.SemaphoreType.DMA,
        pltpu.SemaphoreType.DMA,
    ],
)
def _score_kernel(embs_hbm, wrel_hbm, hidx_hbm, ridx_hbm, tidx_hbm, out_hbm,
                  hidx_v, ridx_v, tidx_v, h_v, r_v, t_v, out_v, sem0, sem1):
    wid = lax.axis_index("s") * NC + lax.axis_index("c")
    pltpu.sync_copy(hidx_hbm.at[wid], hidx_v)
    pltpu.sync_copy(ridx_hbm.at[wid], ridx_v)
    pltpu.sync_copy(tidx_hbm.at[wid], tidx_v)

    last_lane = lax.iota(jnp.int32, L) == (L - 1)
    sems = (sem0, sem1)

    def fire(c):
        s = c % 2
        sem = sems[s]
        return (
            pltpu.async_copy(embs_hbm.at[hidx_v.at[c]], h_v.at[s], sem),
            pltpu.async_copy(wrel_hbm.at[ridx_v.at[c]], r_v.at[s], sem),
            pltpu.async_copy(embs_hbm.at[tidx_v.at[c]], t_v.at[s], sem),
        )

    inflight = fire(0)
    for c in range(NCH):
        for cp in inflight:
            cp.wait()
        if c + 1 < NCH:
            inflight = fire(c + 1)
        s = c % 2

        def row_blk(g, _, c=c, s=s):
            for rr in range(4):
                b = g * 4 + rr
                acc = (h_v[s, b, pl.ds(0, L)] * r_v[s, b, pl.ds(0, L)]
                       * t_v[s, b, pl.ds(0, L)])
                for j in range(1, NJ):
                    acc = acc + (h_v[s, b, pl.ds(j * L, L)]
                                 * r_v[s, b, pl.ds(j * L, L)]
                                 * t_v[s, b, pl.ds(j * L, L)])
                tot = lax.cumsum(acc, axis=0)
                plsc.store_compressed(out_v.at[pl.ds(c * CH + b, L)], tot,
                                      mask=last_lane)
            return 0

        lax.fori_loop(0, CH // 4, row_blk, 0, unroll=False)

    pltpu.sync_copy(out_v.at[pl.ds(0, B_PER_W)], out_hbm.at[wid])


def kernel(embs, sample, w_relation):
    sample = sample.astype(jnp.int32)
    hidx = sample[0].reshape(NW, NCH, CH)
    ridx = sample[1].reshape(NW, NCH, CH)
    tidx = sample[2].reshape(NW, NCH, CH)
    out = _score_kernel(embs, w_relation, hidx, ridx, tidx)
    return out.reshape(B, 1)
